# baseline (speedup 1.0000x reference)
import functools

import jax
import jax.numpy as jnp
from jax import lax
from jax.experimental import pallas as pl
from jax.experimental.pallas import tpu as pltpu
from jax.experimental.pallas import tpu_sc as plsc

B, S, D = 4096, 200, 128
V = 100000
NC, NS, L = 2, 16, 16
NW = NC * NS
BPW = B // NW
HALF = 104
DW = D // 2
NCH = DW // L
VPW = V // NW
PCHUNK = 125

_mesh = plsc.VectorSubcoreMesh(core_axis_name="c", subcore_axis_name="s")
_params = pltpu.CompilerParams(use_tc_tiling_on_sc=False)


@functools.partial(
    pl.kernel,
    mesh=_mesh,
    out_type=jax.ShapeDtypeStruct((V, DW), jnp.int32),
    compiler_params=_params,
    scratch_types=[
        pltpu.VMEM((PCHUNK, D), jnp.float32),
        pltpu.VMEM((PCHUNK, D), jnp.float32),
        pltpu.VMEM((PCHUNK, D), jnp.float32),
        pltpu.VMEM((PCHUNK, D), jnp.float32),
        pltpu.VMEM((PCHUNK, DW), jnp.int32),
        pltpu.VMEM((PCHUNK, DW), jnp.int32),
        pltpu.SemaphoreType.DMA,
        pltpu.SemaphoreType.DMA,
        pltpu.SemaphoreType.DMA,
        pltpu.SemaphoreType.DMA,
        pltpu.SemaphoreType.DMA,
        pltpu.SemaphoreType.DMA,
    ],
)
def _pack_table(table_hbm, out_hbm, in0, in1, in2, in3, pk0, pk1,
                rsem0, rsem1, rsem2, rsem3, wsem0, wsem1):
    wid = lax.axis_index("s") * NC + lax.axis_index("c")
    base = wid * VPW
    nstep = VPW // PCHUNK

    def fetch(step, buf, rsem):
        pltpu.async_copy(
            table_hbm.at[pl.ds(base + step * PCHUNK, PCHUNK)], buf, rsem)

    def fwait(buf, rsem):
        pltpu.make_async_copy(table_hbm.at[pl.ds(0, PCHUNK)], buf, rsem).wait()

    def pack_chunk(step, buf, pk, wsem, wait_prev):
        @pl.when(wait_prev)
        def _():
            pltpu.make_async_copy(pk, out_hbm.at[pl.ds(0, PCHUNK)],
                                  wsem).wait()

        def row_body(q, carry):
            for u in range(5):
                r = q * 5 + u
                for c in range(NCH):
                    flo = buf[r, pl.ds(c * L, L)]
                    fhi = buf[r, pl.ds(DW + c * L, L)]
                    ilo = lax.bitcast_convert_type(flo, jnp.int32)
                    ihi = lax.bitcast_convert_type(fhi, jnp.int32)
                    lo = lax.shift_right_logical(ilo, 16)
                    hi = lax.bitwise_and(ihi, jnp.int32(-65536))
                    pk[r, pl.ds(c * L, L)] = lax.bitwise_or(hi, lo)
            return carry

        lax.fori_loop(0, PCHUNK // 5, row_body, 0)
        pltpu.async_copy(
            pk, out_hbm.at[pl.ds(base + step * PCHUNK, PCHUNK)], wsem)

    ins = [in0, in1, in2, in3]
    rsems = [rsem0, rsem1, rsem2, rsem3]
    pks = [pk0, pk1]
    wsems = [wsem0, wsem1]
    for u in range(4):
        fetch(u, ins[u], rsems[u])

    def quad_body(j, carry):
        s4 = j * 4
        for u in range(4):
            fwait(ins[u], rsems[u])
            pack_chunk(s4 + u, ins[u], pks[u % 2], wsems[u % 2],
                       jnp.logical_or(j > 0, u >= 2))

            @pl.when(s4 + u + 4 < nstep)
            def _():
                fetch(s4 + u + 4, ins[u], rsems[u])

        return carry

    lax.fori_loop(0, nstep // 4, quad_body, 0)
    fwait(in0, rsem0)
    pack_chunk(nstep - 1, in0, pk0, wsem0, True)
    pltpu.make_async_copy(pk0, out_hbm.at[pl.ds(0, PCHUNK)], wsem0).wait()
    pltpu.make_async_copy(pk1, out_hbm.at[pl.ds(0, PCHUNK)], wsem1).wait()


def _fire(table_hbm, tok_v, rbuf, sem, i):
    pltpu.async_copy(table_hbm.at[tok_v.at[pl.ds(i * S, HALF)]],
                     rbuf.at[pl.ds(0, HALF)], sem)
    pltpu.async_copy(table_hbm.at[tok_v.at[pl.ds(i * S + HALF, S - HALF)]],
                     rbuf.at[pl.ds(HALF, S - HALF)], sem)


def _wait(table_hbm, tok_v, rbuf, sem, i):
    pltpu.make_async_copy(table_hbm.at[tok_v.at[pl.ds(i * S, HALF)]],
                          rbuf.at[pl.ds(0, HALF)], sem).wait()
    pltpu.make_async_copy(table_hbm.at[tok_v.at[pl.ds(i * S + HALF, S - HALF)]],
                          rbuf.at[pl.ds(HALF, S - HALF)], sem).wait()


def _accumulate(rbuf, out_v, i):
    def acc_body(q, accs):
        accs = list(accs)
        r = q * 4
        for u in range(4):
            for c in range(NCH):
                w = rbuf[r + u, pl.ds(c * L, L)]
                lo = lax.bitcast_convert_type(
                    lax.shift_left(w, 16), jnp.float32)
                hi = lax.bitcast_convert_type(w, jnp.float32)
                accs[c] = accs[c] + lo
                accs[NCH + c] = accs[NCH + c] + hi
        return tuple(accs)

    accs = lax.fori_loop(
        0, S // 4, acc_body,
        tuple(jnp.zeros((L,), jnp.float32) for _ in range(2 * NCH)))

    for c in range(NCH):
        out_v[i, pl.ds(c * L, L)] = accs[c] * (1.0 / S)
        out_v[i, pl.ds(D // 2 + c * L, L)] = accs[NCH + c] * (1.0 / S)


@functools.partial(
    pl.kernel,
    mesh=_mesh,
    out_type=jax.ShapeDtypeStruct((B, D), jnp.float32),
    compiler_params=_params,
    scratch_types=[
        pltpu.VMEM((BPW * S,), jnp.int32),
        pltpu.VMEM((S, DW), jnp.int32),
        pltpu.VMEM((S, DW), jnp.int32),
        pltpu.VMEM((BPW, D), jnp.float32),
        pltpu.SemaphoreType.DMA,
        pltpu.SemaphoreType.DMA,
    ],
)
def _embed_mean(tokens_hbm, table_hbm, out_hbm,
                tok_v, buf0, buf1, out_v, sem0, sem1):
    wid = lax.axis_index("s") * NC + lax.axis_index("c")
    base = wid * BPW
    pltpu.sync_copy(tokens_hbm.at[pl.ds(base * S, BPW * S)], tok_v)

    _fire(table_hbm, tok_v, buf0, sem0, 0)
    _fire(table_hbm, tok_v, buf1, sem1, 1)

    def pair_body(j, carry):
        i2 = j * 2
        _wait(table_hbm, tok_v, buf0, sem0, i2)
        _accumulate(buf0, out_v, i2)

        @pl.when(i2 + 2 < BPW)
        def _():
            _fire(table_hbm, tok_v, buf0, sem0, i2 + 2)

        _wait(table_hbm, tok_v, buf1, sem1, i2 + 1)
        _accumulate(buf1, out_v, i2 + 1)

        @pl.when(i2 + 3 < BPW)
        def _():
            _fire(table_hbm, tok_v, buf1, sem1, i2 + 3)

        return carry

    lax.fori_loop(0, BPW // 2, pair_body, 0)
    pltpu.sync_copy(out_v, out_hbm.at[pl.ds(base, BPW)])


def kernel(tokens, table):
    tok_flat = tokens.reshape(-1).astype(jnp.int32)
    tw = _pack_table(table)
    return _embed_mean(tok_flat, tw)


# speedup vs baseline: 1.0126x; 1.0126x over previous
import functools

import jax
import jax.numpy as jnp
from jax import lax
from jax.experimental import pallas as pl
from jax.experimental.pallas import tpu as pltpu
from jax.experimental.pallas import tpu_sc as plsc

B, S, D = 4096, 200, 128
V = 100000
NC, NS, L = 2, 16, 16
NW = NC * NS
BPW = B // NW
HALF = 104
DW = D // 2
NCH = DW // L
VPW = V // NW
PCHUNK = 125

_mesh = plsc.VectorSubcoreMesh(core_axis_name="c", subcore_axis_name="s")
_params = pltpu.CompilerParams(use_tc_tiling_on_sc=False)


@functools.partial(
    pl.kernel,
    mesh=_mesh,
    out_type=jax.ShapeDtypeStruct((V, DW), jnp.int32),
    compiler_params=_params,
    scratch_types=[
        pltpu.VMEM((PCHUNK, D), jnp.float32),
        pltpu.VMEM((PCHUNK, D), jnp.float32),
        pltpu.VMEM((PCHUNK, DW), jnp.int32),
        pltpu.VMEM((PCHUNK, DW), jnp.int32),
        pltpu.SemaphoreType.DMA,
        pltpu.SemaphoreType.DMA,
        pltpu.SemaphoreType.DMA,
        pltpu.SemaphoreType.DMA,
    ],
)
def _pack_table(table_hbm, out_hbm, in0, in1, pk0, pk1,
                rsem0, rsem1, wsem0, wsem1):
    wid = lax.axis_index("s") * NC + lax.axis_index("c")
    base = wid * VPW
    nstep = VPW // PCHUNK

    def fetch(step, buf, rsem):
        pltpu.async_copy(
            table_hbm.at[pl.ds(base + step * PCHUNK, PCHUNK)], buf, rsem)

    def fwait(buf, rsem):
        pltpu.make_async_copy(table_hbm.at[pl.ds(0, PCHUNK)], buf, rsem).wait()

    def pack_chunk(step, buf, pk, wsem, wait_prev):
        @pl.when(wait_prev)
        def _():
            pltpu.make_async_copy(pk, out_hbm.at[pl.ds(0, PCHUNK)],
                                  wsem).wait()

        def row_body(q, carry):
            for u in range(5):
                r = q * 5 + u
                for c in range(NCH):
                    flo = buf[r, pl.ds(c * L, L)]
                    fhi = buf[r, pl.ds(DW + c * L, L)]
                    ilo = lax.bitcast_convert_type(flo, jnp.int32)
                    ihi = lax.bitcast_convert_type(fhi, jnp.int32)
                    lo = lax.shift_right_logical(ilo, 16)
                    hi = lax.bitwise_and(ihi, jnp.int32(-65536))
                    pk[r, pl.ds(c * L, L)] = lax.bitwise_or(hi, lo)
            return carry

        lax.fori_loop(0, PCHUNK // 5, row_body, 0)
        pltpu.async_copy(
            pk, out_hbm.at[pl.ds(base + step * PCHUNK, PCHUNK)], wsem)

    fetch(0, in0, rsem0)
    fetch(1, in1, rsem1)

    def pair_body(j, carry):
        s2 = j * 2
        fwait(in0, rsem0)
        pack_chunk(s2, in0, pk0, wsem0, j > 0)

        @pl.when(s2 + 2 < nstep)
        def _():
            fetch(s2 + 2, in0, rsem0)

        fwait(in1, rsem1)
        pack_chunk(s2 + 1, in1, pk1, wsem1, j > 0)

        @pl.when(s2 + 3 < nstep)
        def _():
            fetch(s2 + 3, in1, rsem1)

        return carry

    lax.fori_loop(0, nstep // 2, pair_body, 0)
    fwait(in0, rsem0)
    pack_chunk(nstep - 1, in0, pk0, wsem0, True)
    pltpu.make_async_copy(pk0, out_hbm.at[pl.ds(0, PCHUNK)], wsem0).wait()
    pltpu.make_async_copy(pk1, out_hbm.at[pl.ds(0, PCHUNK)], wsem1).wait()


def _fire(table_hbm, tok_v, rbuf, sem, i):
    pltpu.async_copy(table_hbm.at[tok_v.at[pl.ds(i * S, HALF)]],
                     rbuf.at[pl.ds(0, HALF)], sem)
    pltpu.async_copy(table_hbm.at[tok_v.at[pl.ds(i * S + HALF, S - HALF)]],
                     rbuf.at[pl.ds(HALF, S - HALF)], sem)


def _wait(table_hbm, tok_v, rbuf, sem, i):
    pltpu.make_async_copy(table_hbm.at[tok_v.at[pl.ds(i * S, HALF)]],
                          rbuf.at[pl.ds(0, HALF)], sem).wait()
    pltpu.make_async_copy(table_hbm.at[tok_v.at[pl.ds(i * S + HALF, S - HALF)]],
                          rbuf.at[pl.ds(HALF, S - HALF)], sem).wait()


def _accumulate(rbuf, out_v, i):
    def acc_body(q, accs):
        accs = list(accs)
        r = q * 4
        for u in range(4):
            for c in range(NCH):
                w = rbuf[r + u, pl.ds(c * L, L)]
                lo = lax.bitcast_convert_type(
                    lax.shift_left(w, 16), jnp.float32)
                hi = lax.bitcast_convert_type(w, jnp.float32)
                accs[c] = accs[c] + lo
                accs[NCH + c] = accs[NCH + c] + hi
        return tuple(accs)

    accs = lax.fori_loop(
        0, S // 4, acc_body,
        tuple(jnp.zeros((L,), jnp.float32) for _ in range(2 * NCH)))

    for c in range(NCH):
        out_v[i, pl.ds(c * L, L)] = accs[c] * (1.0 / S)
        out_v[i, pl.ds(D // 2 + c * L, L)] = accs[NCH + c] * (1.0 / S)


@functools.partial(
    pl.kernel,
    mesh=_mesh,
    out_type=jax.ShapeDtypeStruct((B, D), jnp.float32),
    compiler_params=_params,
    scratch_types=[
        pltpu.VMEM((BPW * S,), jnp.int32),
        pltpu.VMEM((S, DW), jnp.int32),
        pltpu.VMEM((S, DW), jnp.int32),
        pltpu.VMEM((BPW, D), jnp.float32),
        pltpu.SemaphoreType.DMA,
        pltpu.SemaphoreType.DMA,
    ],
)
def _embed_mean(tokens_hbm, table_hbm, out_hbm,
                tok_v, buf0, buf1, out_v, sem0, sem1):
    wid = lax.axis_index("s") * NC + lax.axis_index("c")
    base = wid * BPW
    pltpu.sync_copy(tokens_hbm.at[pl.ds(base * S, BPW * S)], tok_v)

    _fire(table_hbm, tok_v, buf0, sem0, 0)
    _fire(table_hbm, tok_v, buf1, sem1, 1)

    def pair_body(j, carry):
        i2 = j * 2
        _wait(table_hbm, tok_v, buf0, sem0, i2)
        _accumulate(buf0, out_v, i2)

        @pl.when(i2 + 2 < BPW)
        def _():
            _fire(table_hbm, tok_v, buf0, sem0, i2 + 2)

        _wait(table_hbm, tok_v, buf1, sem1, i2 + 1)
        _accumulate(buf1, out_v, i2 + 1)

        @pl.when(i2 + 3 < BPW)
        def _():
            _fire(table_hbm, tok_v, buf1, sem1, i2 + 3)

        return carry

    lax.fori_loop(0, BPW // 2, pair_body, 0)
    pltpu.sync_copy(out_v, out_hbm.at[pl.ds(base, BPW)])


def kernel(tokens, table):
    tok_flat = tokens.reshape(-1).astype(jnp.int32)
    tw = _pack_table(table)
    return _embed_mean(tok_flat, tw)
